# Initial kernel scaffold; baseline (speedup 1.0000x reference)
import jax
import jax.numpy as jnp
from jax.experimental import pallas as pl


def kernel(feats, W, b):
    raise NotImplementedError("write your pallas kernel here")


# speedup vs baseline: 3.1312x; 3.1312x over previous
import functools

import jax
import jax.numpy as jnp
from jax import lax
from jax.experimental import pallas as pl
from jax.experimental.pallas import tpu as pltpu
from jax.experimental.pallas import tpu_sc as plsc

_N = 32768
_D = 512
_ROWS_BLK = 1024
_NSUB = 16
_CHUNK = _N // _NSUB
_LANES = 16


def _matvec_body(x_ref, w_ref, o_ref):
    o_ref[...] = jnp.dot(x_ref[...], w_ref[...],
                         preferred_element_type=jnp.float32)


def _matvec(feats, w_col):
    return pl.pallas_call(
        _matvec_body,
        grid=(_N // _ROWS_BLK,),
        in_specs=[
            pl.BlockSpec((_ROWS_BLK, _D), lambda i: (i, 0)),
            pl.BlockSpec((_D, 1), lambda i: (0, 0)),
        ],
        out_specs=pl.BlockSpec((_ROWS_BLK, 1), lambda i: (i, 0)),
        out_shape=jax.ShapeDtypeStruct((_N, 1), jnp.float32),
        compiler_params=pltpu.CompilerParams(
            dimension_semantics=("arbitrary",)),
    )(feats, w_col)


_mesh = plsc.VectorSubcoreMesh(core_axis_name="c", subcore_axis_name="s")


@functools.partial(
    pl.kernel,
    mesh=_mesh,
    out_type=jax.ShapeDtypeStruct((2, _D), jnp.float32),
    scratch_types=[
        pltpu.VMEM((_CHUNK,), jnp.float32),
        pltpu.VMEM((_LANES,), jnp.float32),
        pltpu.VMEM((_LANES,), jnp.int32),
        pltpu.VMEM((_LANES,), jnp.float32),
        pltpu.VMEM((_LANES,), jnp.int32),
        pltpu.VMEM_SHARED((_NSUB * _LANES,), jnp.float32),
        pltpu.VMEM_SHARED((_NSUB * _LANES,), jnp.int32),
        pltpu.VMEM_SHARED((_NSUB * _LANES,), jnp.float32),
        pltpu.VMEM_SHARED((_NSUB * _LANES,), jnp.int32),
        pltpu.VMEM((_NSUB * _LANES,), jnp.float32),
        pltpu.VMEM((_NSUB * _LANES,), jnp.int32),
        pltpu.VMEM((_NSUB * _LANES,), jnp.float32),
        pltpu.VMEM((_NSUB * _LANES,), jnp.int32),
        pltpu.VMEM((_LANES,), jnp.int32),
        pltpu.VMEM((_LANES, _D), jnp.float32),
        pltpu.SemaphoreType.DMA,
    ],
    compiler_params=pltpu.CompilerParams(needs_layout_passes=False),
)
def _sc_top1(d_hbm, feats_hbm, out_hbm, d_v, pvx, pix, pvn, pni,
             shvx, shix, shvn, shni, lvx, lix, lvn, lni, gidx, rows, sem):
    cid = lax.axis_index("c")
    sid = lax.axis_index("s")

    @pl.when(cid == 0)
    def _():
        base = sid * _CHUNK
        pltpu.sync_copy(d_hbm.at[pl.ds(base, _CHUNK)], d_v)
        lanes = lax.iota(jnp.int32, _LANES)
        ninf = jnp.full((_LANES,), -jnp.inf, jnp.float32)
        pinf = jnp.full((_LANES,), jnp.inf, jnp.float32)
        zidx = jnp.zeros((_LANES,), jnp.int32)

        def body(i, carry):
            bvx, bix, bvn, bni = carry
            v = d_v[pl.ds(i * _LANES, _LANES)]
            idx = base + i * _LANES + lanes
            gt = v > bvx
            lt = v < bvn
            return (jnp.where(gt, v, bvx), jnp.where(gt, idx, bix),
                    jnp.where(lt, v, bvn), jnp.where(lt, idx, bni))

        bvx, bix, bvn, bni = lax.fori_loop(
            0, _CHUNK // _LANES, body, (ninf, zidx, pinf, zidx))
        pvx[...] = bvx
        pix[...] = bix
        pvn[...] = bvn
        pni[...] = bni
        off = sid * _LANES
        pltpu.sync_copy(pvx, shvx.at[pl.ds(off, _LANES)])
        pltpu.sync_copy(pix, shix.at[pl.ds(off, _LANES)])
        pltpu.sync_copy(pvn, shvn.at[pl.ds(off, _LANES)])
        pltpu.sync_copy(pni, shni.at[pl.ds(off, _LANES)])
        plsc.subcore_barrier()

        @pl.when(sid == 0)
        def _():
            pltpu.sync_copy(shvx, lvx)
            pltpu.sync_copy(shix, lix)
            pltpu.sync_copy(shvn, lvn)
            pltpu.sync_copy(shni, lni)
            bvx = lvx[pl.ds(0, _LANES)]
            bix = lix[pl.ds(0, _LANES)]
            bvn = lvn[pl.ds(0, _LANES)]
            bni = lni[pl.ds(0, _LANES)]
            for w in range(1, _NSUB):
                v = lvx[pl.ds(w * _LANES, _LANES)]
                ii = lix[pl.ds(w * _LANES, _LANES)]
                gt = v > bvx
                bvx = jnp.where(gt, v, bvx)
                bix = jnp.where(gt, ii, bix)
                v = lvn[pl.ds(w * _LANES, _LANES)]
                ii = lni[pl.ds(w * _LANES, _LANES)]
                lt = v < bvn
                bvn = jnp.where(lt, v, bvn)
                bni = jnp.where(lt, ii, bni)
            for s in (8, 4, 2, 1):
                perm = lanes ^ s
                pvx[...] = bvx
                pix[...] = bix
                pvn[...] = bvn
                pni[...] = bni
                ov = plsc.load_gather(pvx, [perm])
                oi = plsc.load_gather(pix, [perm])
                t = (ov > bvx) | ((ov == bvx) & (oi < bix))
                bvx = jnp.where(t, ov, bvx)
                bix = jnp.where(t, oi, bix)
                ov = plsc.load_gather(pvn, [perm])
                oi = plsc.load_gather(pni, [perm])
                t = (ov < bvn) | ((ov == bvn) & (oi < bni))
                bvn = jnp.where(t, ov, bvn)
                bni = jnp.where(t, oi, bni)
            gv = jnp.where(lanes == 0, bix, jnp.where(lanes == 1, bni, 0))
            gidx[...] = gv
            pltpu.async_copy(feats_hbm.at[gidx], rows, sem).wait()
            pltpu.sync_copy(rows.at[pl.ds(0, 2)], out_hbm)


def kernel(feats, W, b):
    del b
    w_col = (W[0] - W[1]).reshape(_D, 1)
    d = _matvec(feats, w_col).reshape(_N)
    return _sc_top1(d, feats)


# speedup vs baseline: 3.7901x; 1.2104x over previous
import functools

import jax
import jax.numpy as jnp
from jax import lax
from jax.experimental import pallas as pl
from jax.experimental.pallas import tpu as pltpu
from jax.experimental.pallas import tpu_sc as plsc

_N = 32768
_D = 512
_ROWS_BLK = 4096
_NSUB = 16
_CHUNK = _N // _NSUB
_LANES = 16


def _matvec_body(x_ref, w_ref, o_ref):
    o_ref[...] = jnp.dot(x_ref[...], w_ref[...],
                         preferred_element_type=jnp.float32)


def _matvec(feats, w_col):
    return pl.pallas_call(
        _matvec_body,
        grid=(_N // _ROWS_BLK,),
        in_specs=[
            pl.BlockSpec((_ROWS_BLK, _D), lambda i: (i, 0)),
            pl.BlockSpec((_D, 1), lambda i: (0, 0)),
        ],
        out_specs=pl.BlockSpec((_ROWS_BLK, 1), lambda i: (i, 0)),
        out_shape=jax.ShapeDtypeStruct((_N, 1), jnp.float32),
        compiler_params=pltpu.CompilerParams(
            dimension_semantics=("arbitrary",)),
    )(feats, w_col)


_mesh = plsc.VectorSubcoreMesh(core_axis_name="c", subcore_axis_name="s")


@functools.partial(
    pl.kernel,
    mesh=_mesh,
    out_type=jax.ShapeDtypeStruct((2, _D), jnp.float32),
    scratch_types=[
        pltpu.VMEM((_CHUNK,), jnp.float32),
        pltpu.VMEM((_LANES,), jnp.float32),
        pltpu.VMEM((_LANES,), jnp.int32),
        pltpu.VMEM((_LANES,), jnp.float32),
        pltpu.VMEM((_LANES,), jnp.int32),
        pltpu.VMEM_SHARED((_NSUB * _LANES,), jnp.float32),
        pltpu.VMEM_SHARED((_NSUB * _LANES,), jnp.int32),
        pltpu.VMEM_SHARED((_NSUB * _LANES,), jnp.float32),
        pltpu.VMEM_SHARED((_NSUB * _LANES,), jnp.int32),
        pltpu.VMEM((_NSUB * _LANES,), jnp.float32),
        pltpu.VMEM((_NSUB * _LANES,), jnp.int32),
        pltpu.VMEM((_NSUB * _LANES,), jnp.float32),
        pltpu.VMEM((_NSUB * _LANES,), jnp.int32),
        pltpu.VMEM((_LANES,), jnp.int32),
        pltpu.VMEM((_LANES, _D), jnp.float32),
        pltpu.SemaphoreType.DMA,
    ],
    compiler_params=pltpu.CompilerParams(needs_layout_passes=False),
)
def _sc_top1(d_hbm, feats_hbm, out_hbm, d_v, pvx, pix, pvn, pni,
             shvx, shix, shvn, shni, lvx, lix, lvn, lni, gidx, rows, sem):
    cid = lax.axis_index("c")
    sid = lax.axis_index("s")

    @pl.when(cid == 0)
    def _():
        base = sid * _CHUNK
        pltpu.sync_copy(d_hbm.at[pl.ds(base, _CHUNK)], d_v)
        lanes = lax.iota(jnp.int32, _LANES)
        ninf = jnp.full((_LANES,), -jnp.inf, jnp.float32)
        pinf = jnp.full((_LANES,), jnp.inf, jnp.float32)
        zidx = jnp.zeros((_LANES,), jnp.int32)

        def body(i, carry):
            bvx, bix, bvn, bni = carry
            v = d_v[pl.ds(i * _LANES, _LANES)]
            idx = base + i * _LANES + lanes
            gt = v > bvx
            lt = v < bvn
            return (jnp.where(gt, v, bvx), jnp.where(gt, idx, bix),
                    jnp.where(lt, v, bvn), jnp.where(lt, idx, bni))

        bvx, bix, bvn, bni = lax.fori_loop(
            0, _CHUNK // _LANES, body, (ninf, zidx, pinf, zidx))
        pvx[...] = bvx
        pix[...] = bix
        pvn[...] = bvn
        pni[...] = bni
        off = sid * _LANES
        pltpu.sync_copy(pvx, shvx.at[pl.ds(off, _LANES)])
        pltpu.sync_copy(pix, shix.at[pl.ds(off, _LANES)])
        pltpu.sync_copy(pvn, shvn.at[pl.ds(off, _LANES)])
        pltpu.sync_copy(pni, shni.at[pl.ds(off, _LANES)])
        plsc.subcore_barrier()

        @pl.when(sid == 0)
        def _():
            pltpu.sync_copy(shvx, lvx)
            pltpu.sync_copy(shix, lix)
            pltpu.sync_copy(shvn, lvn)
            pltpu.sync_copy(shni, lni)
            bvx = lvx[pl.ds(0, _LANES)]
            bix = lix[pl.ds(0, _LANES)]
            bvn = lvn[pl.ds(0, _LANES)]
            bni = lni[pl.ds(0, _LANES)]
            for w in range(1, _NSUB):
                v = lvx[pl.ds(w * _LANES, _LANES)]
                ii = lix[pl.ds(w * _LANES, _LANES)]
                gt = v > bvx
                bvx = jnp.where(gt, v, bvx)
                bix = jnp.where(gt, ii, bix)
                v = lvn[pl.ds(w * _LANES, _LANES)]
                ii = lni[pl.ds(w * _LANES, _LANES)]
                lt = v < bvn
                bvn = jnp.where(lt, v, bvn)
                bni = jnp.where(lt, ii, bni)
            for s in (8, 4, 2, 1):
                perm = lanes ^ s
                pvx[...] = bvx
                pix[...] = bix
                pvn[...] = bvn
                pni[...] = bni
                ov = plsc.load_gather(pvx, [perm])
                oi = plsc.load_gather(pix, [perm])
                t = (ov > bvx) | ((ov == bvx) & (oi < bix))
                bvx = jnp.where(t, ov, bvx)
                bix = jnp.where(t, oi, bix)
                ov = plsc.load_gather(pvn, [perm])
                oi = plsc.load_gather(pni, [perm])
                t = (ov < bvn) | ((ov == bvn) & (oi < bni))
                bvn = jnp.where(t, ov, bvn)
                bni = jnp.where(t, oi, bni)
            gv = jnp.where(lanes == 0, bix, jnp.where(lanes == 1, bni, 0))
            gidx[...] = gv
            pltpu.async_copy(feats_hbm.at[gidx], rows, sem).wait()
            pltpu.sync_copy(rows.at[pl.ds(0, 2)], out_hbm)


def kernel(feats, W, b):
    del b
    w_col = (W[0] - W[1]).reshape(_D, 1)
    d = _matvec(feats, w_col).reshape(_N)
    return _sc_top1(d, feats)
